# baseline (speedup 1.0000x reference)
import functools

import jax
import jax.numpy as jnp
from jax import lax
from jax.experimental import pallas as pl
from jax.experimental.pallas import tpu as pltpu
from jax.experimental.pallas import tpu_sc as plsc

NC, NS, LANES = 2, 16, 16
CH = 128
BLK = 1024


def _sc_hist_body(cpt, n_pad, ei, deg_out, ones_v, dslab, zeros_v, deg_sp, sem):
    c = lax.axis_index("c")
    s = lax.axis_index("s")
    w = c * NS + s
    rpt = n_pad // NS

    for j in range(CH // LANES):
        ones_v[pl.ds(j * LANES, LANES)] = jnp.ones((LANES,), jnp.float32)

    def zero_body(i, _):
        zeros_v[pl.ds(i * LANES, LANES)] = jnp.zeros((LANES,), jnp.float32)
        return 0
    lax.fori_loop(0, rpt // LANES, zero_body, 0)

    pltpu.sync_copy(zeros_v, deg_sp.at[pl.ds(s * rpt, rpt)])
    plsc.subcore_barrier()

    pltpu.sync_copy(ei.at[1, pl.ds(w * cpt, cpt), :], dslab)
    descs = [
        pltpu.async_copy(ones_v, deg_sp.at[dslab.at[k]], sem, add=True)
        for k in range(cpt)
    ]
    for d in descs:
        d.wait()

    plsc.subcore_barrier()
    pltpu.sync_copy(deg_sp.at[pl.ds(s * rpt, rpt)],
                    deg_out.at[c, pl.ds(s * rpt, rpt)])


def _sc_agg_body(ng, gsz, n_pad, ei, hs_hbm, s_out,
                 gbuf0, gbuf1, sidx, didx, acc_sp, sem0, sem1):
    c = lax.axis_index("c")
    s = lax.axis_index("s")
    w = c * NS + s
    cpt = ng * gsz
    rpt = n_pad // NS

    def zero_row(r, _):
        for j in range(CH // LANES):
            gbuf0[r, pl.ds(j * LANES, LANES)] = jnp.zeros((LANES,), jnp.float32)
        return 0
    lax.fori_loop(0, CH, zero_row, 0)
    for k in range(rpt // CH):
        pltpu.sync_copy(gbuf0, acc_sp.at[pl.ds(s * rpt + k * CH, CH), :])
    plsc.subcore_barrier()

    for gi in range(ng):
        base = w * cpt + gi * gsz
        pltpu.sync_copy(ei.at[0, pl.ds(base, gsz), :], sidx)
        pltpu.sync_copy(ei.at[1, pl.ds(base, gsz), :], didx)
        pltpu.async_copy(hs_hbm.at[sidx.at[0]], gbuf0, sem0)

        def body(j, _):
            k0 = 2 * j
            k1 = k0 + 1
            pltpu.async_copy(hs_hbm.at[sidx.at[k1]], gbuf1, sem1)
            pltpu.make_async_copy(hs_hbm.at[sidx.at[k0]], gbuf0, sem0).wait()
            pltpu.sync_copy(gbuf0, acc_sp.at[didx.at[k0]], add=True)

            @pl.when(j < gsz // 2 - 1)
            def _():
                pltpu.async_copy(hs_hbm.at[sidx.at[k0 + 2]], gbuf0, sem0)

            pltpu.make_async_copy(hs_hbm.at[sidx.at[k1]], gbuf1, sem1).wait()
            pltpu.sync_copy(gbuf1, acc_sp.at[didx.at[k1]], add=True)
            return 0
        lax.fori_loop(0, gsz // 2, body, 0)

    plsc.subcore_barrier()
    for k in range(rpt // CH):
        r0 = s * rpt + k * CH
        pltpu.sync_copy(acc_sp.at[pl.ds(r0, CH), :],
                        s_out.at[c, pl.ds(r0, CH), :])


def _dinv_col(deg_ref, i, m):
    d = deg_ref[0, pl.ds(i * BLK, BLK)] + deg_ref[1, pl.ds(i * BLK, BLK)]
    return lax.rsqrt(d + 1.0)[:, None]


def _tca_body(x_ref, w_ref, h_ref):
    h_ref[...] = jnp.dot(x_ref[...], w_ref[...],
                         preferred_element_type=jnp.float32)


def _tcb_body(h_ref, deg_ref, hs_ref):
    i = pl.program_id(0)
    hs_ref[...] = h_ref[...] * _dinv_col(deg_ref, i, BLK)


def _tc2_body(s_ref, hs_ref, deg_ref, b1_ref, wc_ref, bc_ref, y_ref, z_ref):
    i = pl.program_id(0)
    dinv = _dinv_col(deg_ref, i, BLK)
    z = dinv * (s_ref[0] + s_ref[1] + hs_ref[...]) + b1_ref[...]
    z_ref[...] = z
    y_ref[...] = (jnp.dot(z, wc_ref[...], preferred_element_type=jnp.float32)
                  + bc_ref[...])


def kernel(x, edge_index, W1, b1, Wc, bc):
    n, f_in = x.shape
    h_dim = W1.shape[1]
    e = edge_index.shape[1]

    n_pad = ((n + NS * CH) // (NS * CH)) * (NS * CH)
    ng = 2
    cpt = -(-e // (NC * NS * CH))
    cpt = ((cpt + 2 * ng - 1) // (2 * ng)) * (2 * ng)
    gsz = cpt // ng
    nch = NC * NS * cpt
    pad = nch * CH - e

    ar = jnp.arange(pad, dtype=jnp.int32)
    fill = jnp.stack([ar % jnp.int32(n), n + ar % jnp.int32(n_pad - n)])
    ei = jnp.concatenate([edge_index.astype(jnp.int32), fill], axis=1)
    ei = ei.reshape(2, nch, CH)

    mesh = plsc.VectorSubcoreMesh(core_axis_name="c", subcore_axis_name="s",
                                  num_cores=NC, num_subcores=NS)

    sc_hist = pl.kernel(
        functools.partial(_sc_hist_body, cpt, n_pad),
        out_type=jax.ShapeDtypeStruct((NC, n_pad), jnp.float32),
        mesh=mesh,
        scratch_types=[
            pltpu.VMEM((CH,), jnp.float32),
            pltpu.VMEM((cpt, CH), jnp.int32),
            pltpu.VMEM((n_pad // NS,), jnp.float32),
            pltpu.VMEM_SHARED((n_pad,), jnp.float32),
            pltpu.SemaphoreType.DMA,
        ],
    )
    deg = sc_hist(ei)

    grid = -(-n // BLK)

    h = pl.pallas_call(
        _tca_body,
        grid=(grid,),
        in_specs=[
            pl.BlockSpec((BLK, f_in), lambda i: (i, 0)),
            pl.BlockSpec((f_in, h_dim), lambda i: (0, 0)),
        ],
        out_specs=pl.BlockSpec((BLK, h_dim), lambda i: (i, 0)),
        out_shape=jax.ShapeDtypeStruct((n, h_dim), jnp.float32),
    )(x, W1)

    hs = pl.pallas_call(
        _tcb_body,
        grid=(grid,),
        in_specs=[
            pl.BlockSpec((BLK, h_dim), lambda i: (i, 0)),
            pl.BlockSpec((NC, n_pad), lambda i: (0, 0)),
        ],
        out_specs=pl.BlockSpec((BLK, h_dim), lambda i: (i, 0)),
        out_shape=jax.ShapeDtypeStruct((n, h_dim), jnp.float32),
    )(h, deg)

    sc_agg = pl.kernel(
        functools.partial(_sc_agg_body, ng, gsz, n_pad),
        out_type=jax.ShapeDtypeStruct((NC, n_pad, h_dim), jnp.float32),
        mesh=mesh,
        scratch_types=[
            pltpu.VMEM((CH, h_dim), jnp.float32),
            pltpu.VMEM((CH, h_dim), jnp.float32),
            pltpu.VMEM((gsz, CH), jnp.int32),
            pltpu.VMEM((gsz, CH), jnp.int32),
            pltpu.VMEM_SHARED((n_pad, h_dim), jnp.float32),
            pltpu.SemaphoreType.DMA,
            pltpu.SemaphoreType.DMA,
        ],
    )
    s_part = sc_agg(ei, hs)

    y, z = pl.pallas_call(
        _tc2_body,
        grid=(grid,),
        in_specs=[
            pl.BlockSpec((NC, BLK, h_dim), lambda i: (0, i, 0)),
            pl.BlockSpec((BLK, h_dim), lambda i: (i, 0)),
            pl.BlockSpec((NC, n_pad), lambda i: (0, 0)),
            pl.BlockSpec((1, h_dim), lambda i: (0, 0)),
            pl.BlockSpec((h_dim, 1), lambda i: (0, 0)),
            pl.BlockSpec((1, 1), lambda i: (0, 0)),
        ],
        out_specs=[
            pl.BlockSpec((BLK, 1), lambda i: (i, 0)),
            pl.BlockSpec((BLK, h_dim), lambda i: (i, 0)),
        ],
        out_shape=[
            jax.ShapeDtypeStruct((n, 1), jnp.float32),
            jax.ShapeDtypeStruct((n, h_dim), jnp.float32),
        ],
    )(s_part, hs, deg, b1.reshape(1, h_dim), Wc, bc.reshape(1, 1))

    return (y, z)


# speedup vs baseline: 1.0272x; 1.0272x over previous
import functools

import numpy as np

import jax
import jax.numpy as jnp
from jax import lax
from jax.experimental import pallas as pl
from jax.experimental.pallas import tpu as pltpu
from jax.experimental.pallas import tpu_sc as plsc

NC, NS, LANES = 2, 16, 16
CH = 128
BLK = 1024


def _sc_hist_body(cpt, n_pad, ei, deg_out, ones_v, dslab, zeros_v, deg_sp, sem):
    c = lax.axis_index("c")
    s = lax.axis_index("s")
    w = c * NS + s
    rpt = n_pad // NS

    for j in range(CH // LANES):
        ones_v[pl.ds(j * LANES, LANES)] = jnp.ones((LANES,), jnp.float32)

    def zero_body(i, _):
        zeros_v[pl.ds(i * LANES, LANES)] = jnp.zeros((LANES,), jnp.float32)
        return 0
    lax.fori_loop(0, rpt // LANES, zero_body, 0)

    pltpu.sync_copy(zeros_v, deg_sp.at[pl.ds(s * rpt, rpt)])
    plsc.subcore_barrier()

    pltpu.sync_copy(ei.at[1, pl.ds(w * cpt, cpt), :], dslab)
    descs = [
        pltpu.async_copy(ones_v, deg_sp.at[dslab.at[k]], sem, add=True)
        for k in range(cpt)
    ]
    for d in descs:
        d.wait()

    plsc.subcore_barrier()
    pltpu.sync_copy(deg_sp.at[pl.ds(s * rpt, rpt)],
                    deg_out.at[c, pl.ds(s * rpt, rpt)])


def _sc_agg_body(ng, gsz, n_pad, ei, hs_hbm, s_out,
                 gbuf0, gbuf1, sidx, didx, acc_sp, sem0, sem1):
    c = lax.axis_index("c")
    s = lax.axis_index("s")
    w = c * NS + s
    cpt = ng * gsz
    rpt = n_pad // NS

    def zero_row(r, _):
        for j in range(CH // LANES):
            gbuf0[r, pl.ds(j * LANES, LANES)] = jnp.zeros((LANES,), jnp.float32)
        return 0
    lax.fori_loop(0, CH, zero_row, 0)
    for k in range(rpt // CH):
        pltpu.sync_copy(gbuf0, acc_sp.at[pl.ds(s * rpt + k * CH, CH), :])
    plsc.subcore_barrier()

    for gi in range(ng):
        base = w * cpt + gi * gsz
        pltpu.sync_copy(ei.at[0, pl.ds(base, gsz), :], sidx)
        pltpu.sync_copy(ei.at[1, pl.ds(base, gsz), :], didx)
        pltpu.async_copy(hs_hbm.at[sidx.at[0]], gbuf0, sem0)

        def body(j, _):
            k0 = 2 * j
            k1 = k0 + 1
            pltpu.async_copy(hs_hbm.at[sidx.at[k1]], gbuf1, sem1)
            pltpu.make_async_copy(hs_hbm.at[sidx.at[k0]], gbuf0, sem0).wait()
            pltpu.sync_copy(gbuf0, acc_sp.at[didx.at[k0]], add=True)

            @pl.when(j < gsz // 2 - 1)
            def _():
                pltpu.async_copy(hs_hbm.at[sidx.at[k0 + 2]], gbuf0, sem0)

            pltpu.make_async_copy(hs_hbm.at[sidx.at[k1]], gbuf1, sem1).wait()
            pltpu.sync_copy(gbuf1, acc_sp.at[didx.at[k1]], add=True)
            return 0
        lax.fori_loop(0, gsz // 2, body, 0)

    plsc.subcore_barrier()
    for k in range(rpt // CH):
        r0 = s * rpt + k * CH
        pltpu.sync_copy(acc_sp.at[pl.ds(r0, CH), :],
                        s_out.at[c, pl.ds(r0, CH), :])


def _dinv_col(deg_ref, i, m):
    d = deg_ref[0, pl.ds(i * BLK, BLK)] + deg_ref[1, pl.ds(i * BLK, BLK)]
    return lax.rsqrt(d + 1.0)[:, None]


def _tc1_body(x_ref, deg_ref, w_ref, hs_ref):
    i = pl.program_id(0)
    h = jnp.dot(x_ref[...], w_ref[...], preferred_element_type=jnp.float32)
    hs_ref[...] = h * _dinv_col(deg_ref, i, BLK)


def _tc2_body(s_ref, hs_ref, deg_ref, b1_ref, wc_ref, bc_ref, y_ref, z_ref):
    i = pl.program_id(0)
    dinv = _dinv_col(deg_ref, i, BLK)
    z = dinv * (s_ref[0] + s_ref[1] + hs_ref[...]) + b1_ref[...]
    z_ref[...] = z
    y_ref[...] = (jnp.dot(z, wc_ref[...], preferred_element_type=jnp.float32)
                  + bc_ref[...])


def kernel(x, edge_index, W1, b1, Wc, bc):
    n, f_in = x.shape
    h_dim = W1.shape[1]
    e = edge_index.shape[1]

    n_pad = ((n + NS * CH) // (NS * CH)) * (NS * CH)
    ng = 2
    cpt = -(-e // (NC * NS * CH))
    cpt = ((cpt + 2 * ng - 1) // (2 * ng)) * (2 * ng)
    gsz = cpt // ng
    nch = NC * NS * cpt
    pad = nch * CH - e

    ar = np.arange(pad, dtype=np.int32)
    fill = jnp.asarray(np.stack([ar % n, n + ar % (n_pad - n)]), jnp.int32)
    ei = jnp.concatenate([edge_index.astype(jnp.int32), fill], axis=1)
    ei = ei.reshape(2, nch, CH)

    mesh = plsc.VectorSubcoreMesh(core_axis_name="c", subcore_axis_name="s",
                                  num_cores=NC, num_subcores=NS)

    sc_hist = pl.kernel(
        functools.partial(_sc_hist_body, cpt, n_pad),
        out_type=jax.ShapeDtypeStruct((NC, n_pad), jnp.float32),
        mesh=mesh,
        scratch_types=[
            pltpu.VMEM((CH,), jnp.float32),
            pltpu.VMEM((cpt, CH), jnp.int32),
            pltpu.VMEM((n_pad // NS,), jnp.float32),
            pltpu.VMEM_SHARED((n_pad,), jnp.float32),
            pltpu.SemaphoreType.DMA,
        ],
    )
    deg = sc_hist(ei)

    grid = -(-n // BLK)

    hs = pl.pallas_call(
        _tc1_body,
        grid=(grid,),
        in_specs=[
            pl.BlockSpec((BLK, f_in), lambda i: (i, 0)),
            pl.BlockSpec((NC, n_pad), lambda i: (0, 0)),
            pl.BlockSpec((f_in, h_dim), lambda i: (0, 0)),
        ],
        out_specs=pl.BlockSpec((BLK, h_dim), lambda i: (i, 0)),
        out_shape=jax.ShapeDtypeStruct((n, h_dim), jnp.float32),
    )(x, deg, W1)

    sc_agg = pl.kernel(
        functools.partial(_sc_agg_body, ng, gsz, n_pad),
        out_type=jax.ShapeDtypeStruct((NC, n_pad, h_dim), jnp.float32),
        mesh=mesh,
        scratch_types=[
            pltpu.VMEM((CH, h_dim), jnp.float32),
            pltpu.VMEM((CH, h_dim), jnp.float32),
            pltpu.VMEM((gsz, CH), jnp.int32),
            pltpu.VMEM((gsz, CH), jnp.int32),
            pltpu.VMEM_SHARED((n_pad, h_dim), jnp.float32),
            pltpu.SemaphoreType.DMA,
            pltpu.SemaphoreType.DMA,
        ],
    )
    s_part = sc_agg(ei, hs)

    y, z = pl.pallas_call(
        _tc2_body,
        grid=(grid,),
        in_specs=[
            pl.BlockSpec((NC, BLK, h_dim), lambda i: (0, i, 0)),
            pl.BlockSpec((BLK, h_dim), lambda i: (i, 0)),
            pl.BlockSpec((NC, n_pad), lambda i: (0, 0)),
            pl.BlockSpec((1, h_dim), lambda i: (0, 0)),
            pl.BlockSpec((h_dim, 1), lambda i: (0, 0)),
            pl.BlockSpec((1, 1), lambda i: (0, 0)),
        ],
        out_specs=[
            pl.BlockSpec((BLK, 1), lambda i: (i, 0)),
            pl.BlockSpec((BLK, h_dim), lambda i: (i, 0)),
        ],
        out_shape=[
            jax.ShapeDtypeStruct((n, 1), jnp.float32),
            jax.ShapeDtypeStruct((n, h_dim), jnp.float32),
        ],
    )(s_part, hs, deg, b1.reshape(1, h_dim), Wc, bc.reshape(1, 1))

    return (y, z)
